# Initial kernel scaffold; baseline (speedup 1.0000x reference)
import jax
import jax.numpy as jnp
from jax.experimental import pallas as pl


def kernel(pred_logits, pred_boxes, pos_maps, target_sizes):
    raise NotImplementedError("write your pallas kernel here")


# speedup vs baseline: 14.1034x; 14.1034x over previous
import functools

import jax
import jax.numpy as jnp
from jax import lax
from jax.experimental import pallas as pl
from jax.experimental.pallas import tpu as pltpu
from jax.experimental.pallas import tpu_sc as plsc

B, Q, C, L = 32, 900, 256, 92
QP = 912
RP = 96
NSEL = 300
SP = 304
NCHUNK = QP // 16
FLAT = QP * RP


def _score_body(logits_ref, pm_ref, out_ref, rm_ref):
    sig = jax.nn.sigmoid(logits_ref[0])
    pm = pm_ref[0]
    acc = lax.dot_general(sig, pm, (((1,), (1,)), ((), ())),
                          preferred_element_type=jnp.float32)
    row = lax.broadcasted_iota(jnp.int32, (QP, RP), 0)
    col = lax.broadcasted_iota(jnp.int32, (QP, RP), 1)
    pad = jnp.logical_or(row >= Q, col >= L)
    padded = jnp.where(pad, -1.0, acc)
    out_ref[0] = padded
    rm_ref[0, 0] = jnp.max(padded, axis=1)


def _scores_tc(logits_pad, pos_maps_pad):
    return pl.pallas_call(
        _score_body,
        grid=(B,),
        in_specs=[
            pl.BlockSpec((1, QP, C), lambda b: (b, 0, 0)),
            pl.BlockSpec((1, RP, C), lambda b: (b, 0, 0)),
        ],
        out_specs=[
            pl.BlockSpec((1, QP, RP), lambda b: (b, 0, 0)),
            pl.BlockSpec((1, 1, QP), lambda b: (b, 0, 0)),
        ],
        out_shape=[
            jax.ShapeDtypeStruct((B, QP, RP), jnp.float32),
            jax.ShapeDtypeStruct((B, 1, QP), jnp.float32),
        ],
        compiler_params=pltpu.CompilerParams(
            dimension_semantics=("arbitrary",)),
    )(logits_pad, pos_maps_pad)


def _sc_body(prob_hbm, rowmax_hbm, boxes_hbm, scale_hbm,
             scores_o, labels_o, boxes_o, tk_o,
             prob_v, boxes_v, scale_v, rowmax_v, chunkmax_v,
             score_st, label_st, qidx_st, boxes_st, tk_st):
    b = lax.axis_index("c") * 16 + lax.axis_index("s")
    lanes = lax.iota(jnp.int32, 16)
    lane0 = lanes == 0
    fz = jnp.zeros((16,), jnp.float32)
    iz = jnp.zeros((16,), jnp.int32)

    pltpu.sync_copy(prob_hbm.at[b], prob_v)
    pltpu.sync_copy(rowmax_hbm.at[b], rowmax_v)
    pltpu.sync_copy(boxes_hbm.at[b], boxes_v)
    pltpu.sync_copy(scale_hbm.at[b], scale_v)

    sv = scale_v[...]
    _gd = lax.GatherDimensionNumbers(offset_dims=(), collapsed_slice_dims=(0,),
                                     start_index_map=(0,))
    def _vgather(vec, idx):
        return lax.gather(vec, idx[:, None], _gd, (1,),
                          mode=lax.GatherScatterMode.PROMISE_IN_BOUNDS)
    sw = _vgather(sv, iz)
    sh = _vgather(sv, iz + 1)

    def init_chunks(j, _):
        m = rowmax_v[pl.ds(j * 16, 16)]
        plsc.store_scatter(chunkmax_v, [jnp.full((16,), j, jnp.int32)],
                           jnp.full((16,), jnp.max(m)), mask=lane0)
        return 0
    lax.fori_loop(0, NCHUNK, init_chunks, 0)
    cpad = lax.iota(jnp.int32, 16) + 48
    plsc.store_scatter(chunkmax_v, [cpad], jnp.full((16,), -3.0),
                       mask=cpad >= NCHUNK)
    qidx_st[pl.ds(288, 16)] = iz

    lanes_f = lanes.astype(jnp.float32)
    BIGF = jnp.full((16,), 1e9)

    def extract(i, _):
        mv, mi = jnp.full((16,), -4.0), fz
        for j in range(4):
            v = chunkmax_v[pl.ds(j * 16, 16)]
            upd = v > mv
            mv = jnp.where(upd, v, mv)
            mi = jnp.where(upd, j * 16.0 + lanes_f, mi)
        gmax = jnp.max(mv)
        gmv = jnp.full((16,), gmax)
        jsv = jnp.full((16,), jnp.min(jnp.where(mv == gmv, mi, BIGF))
                       ).astype(jnp.int32)
        rv = plsc.load_gather(rowmax_v, [jsv * 16 + lanes])
        qsv = jsv * 16 + plsc.all_reduce_ffs(rv == gmv)
        basev = qsv * RP
        mv2, mi2 = jnp.full((16,), -4.0), fz
        for cc in range(RP // 16):
            v = plsc.load_gather(prob_v, [basev + cc * 16 + lanes])
            upd = v > mv2
            mv2 = jnp.where(upd, v, mv2)
            mi2 = jnp.where(upd, cc * 16.0 + lanes_f, mi2)
        lsv = jnp.full((16,), jnp.min(jnp.where(mv2 == gmv, mi2, BIGF))
                       ).astype(jnp.int32)

        bi = jnp.full((16,), i, jnp.int32)
        plsc.store_scatter(score_st, [bi], gmv, mask=lane0)
        plsc.store_scatter(label_st, [bi], lsv, mask=lane0)
        plsc.store_scatter(qidx_st, [bi], qsv, mask=lane0)

        plsc.store_scatter(prob_v, [basev + lsv],
                           jnp.full((16,), -1.0), mask=lane0)
        nm = jnp.full((16,), -1.0)
        for cc in range(RP // 16):
            nm = jnp.maximum(
                nm, plsc.load_gather(prob_v, [basev + cc * 16 + lanes]))
        plsc.store_scatter(rowmax_v, [qsv],
                           jnp.full((16,), jnp.max(nm)), mask=lane0)
        rv2 = plsc.load_gather(rowmax_v, [jsv * 16 + lanes])
        plsc.store_scatter(chunkmax_v, [jsv],
                           jnp.full((16,), jnp.max(rv2)), mask=lane0)
        return 0
    lax.fori_loop(0, NSEL, extract, 0)

    def box_chunk(j, _):
        k16 = j * 16 + lanes
        valid = k16 < NSEL
        q16 = qidx_st[pl.ds(j * 16, 16)]
        cx = plsc.load_gather(boxes_v, [q16 * 4])
        cy = plsc.load_gather(boxes_v, [q16 * 4 + 1])
        w = plsc.load_gather(boxes_v, [q16 * 4 + 2])
        h = plsc.load_gather(boxes_v, [q16 * 4 + 3])
        plsc.store_scatter(boxes_st, [k16 * 4], (cx - 0.5 * w) * sw,
                           mask=valid)
        plsc.store_scatter(boxes_st, [k16 * 4 + 1], (cy - 0.5 * h) * sh,
                           mask=valid)
        plsc.store_scatter(boxes_st, [k16 * 4 + 2], (cx + 0.5 * w) * sw,
                           mask=valid)
        plsc.store_scatter(boxes_st, [k16 * 4 + 3], (cy + 0.5 * h) * sh,
                           mask=valid)
        return 0
    lax.fori_loop(0, SP // 16, box_chunk, 0)

    def restore(j, _):
        k16 = j * 16 + lanes
        valid = k16 < NSEL
        v = score_st[pl.ds(j * 16, 16)]
        q16 = qidx_st[pl.ds(j * 16, 16)]
        l16 = label_st[pl.ds(j * 16, 16)]
        plsc.store_scatter(prob_v, [q16 * RP + l16], v, mask=valid)
        return 0
    lax.fori_loop(0, SP // 16, restore, 0)

    def tkrow(k, _):
        qv = plsc.load_gather(qidx_st, [jnp.full((16,), k, jnp.int32)])
        basev = qv * RP + lanes
        for cc in range(RP // 16):
            tk_st[pl.ds(k * RP + cc * 16, 16)] = \
                plsc.load_gather(prob_v, [basev + cc * 16])
        return 0
    lax.fori_loop(0, NSEL, tkrow, 0)

    pltpu.sync_copy(score_st, scores_o.at[b])
    pltpu.sync_copy(label_st, labels_o.at[b])
    pltpu.sync_copy(boxes_st, boxes_o.at[b])
    pltpu.sync_copy(tk_st, tk_o.at[b])


_sc_topk = functools.partial(
    pl.kernel,
    out_type=[
        jax.ShapeDtypeStruct((B, SP), jnp.float32),
        jax.ShapeDtypeStruct((B, SP), jnp.int32),
        jax.ShapeDtypeStruct((B, SP * 4), jnp.float32),
        jax.ShapeDtypeStruct((B, NSEL * RP), jnp.float32),
    ],
    mesh=plsc.VectorSubcoreMesh(core_axis_name="c", subcore_axis_name="s"),
    compiler_params=pltpu.CompilerParams(needs_layout_passes=False),
    scratch_types=[
        pltpu.VMEM((FLAT,), jnp.float32),
        pltpu.VMEM((Q * 4,), jnp.float32),
        pltpu.VMEM((16,), jnp.float32),
        pltpu.VMEM((QP,), jnp.float32),
        pltpu.VMEM((64,), jnp.float32),
        pltpu.VMEM((SP,), jnp.float32),
        pltpu.VMEM((SP,), jnp.int32),
        pltpu.VMEM((SP,), jnp.int32),
        pltpu.VMEM((SP * 4,), jnp.float32),
        pltpu.VMEM((NSEL * RP,), jnp.float32),
    ],
)(_sc_body)


def kernel(pred_logits, pred_boxes, pos_maps, target_sizes):
    logits_pad = jnp.pad(pred_logits, ((0, 0), (0, QP - Q), (0, 0)))
    pm_pad = jnp.pad(pos_maps, ((0, 0), (0, RP - L), (0, 0)))
    prob, rowmax = _scores_tc(logits_pad, pm_pad)
    prob = prob.reshape(B, FLAT)
    rowmax = rowmax.reshape(B, QP)

    w = target_sizes[:, 1].astype(jnp.float32)
    h = target_sizes[:, 0].astype(jnp.float32)
    scale = jnp.concatenate(
        [w[:, None], h[:, None], jnp.zeros((B, 14), jnp.float32)], axis=1)

    scores_p, labels_p, boxes_p, tk_p = _sc_topk(
        prob, rowmax, pred_boxes.reshape(B, Q * 4), scale)

    scores = scores_p[:, :NSEL]
    labels = labels_p[:, :NSEL]
    boxes = boxes_p[:, :NSEL * 4].reshape(B, NSEL, 4)
    topk_prob = tk_p.reshape(B, NSEL, RP)[:, :, :L]
    return scores, labels, boxes, topk_prob


# speedup vs baseline: 14.1155x; 1.0009x over previous
import functools

import jax
import jax.numpy as jnp
from jax import lax
from jax.experimental import pallas as pl
from jax.experimental.pallas import tpu as pltpu
from jax.experimental.pallas import tpu_sc as plsc

B, Q, C, L = 32, 900, 256, 92
QP = 912
RP = 96
NSEL = 300
SP = 304
NCHUNK = QP // 16
FLAT = QP * RP


def _score_body(logits_ref, pm_ref, out_ref, rm_ref):
    sig = jax.nn.sigmoid(logits_ref[0])
    pm = pm_ref[0]
    acc = lax.dot_general(sig, pm, (((1,), (1,)), ((), ())),
                          preferred_element_type=jnp.float32)
    row = lax.broadcasted_iota(jnp.int32, (QP, RP), 0)
    col = lax.broadcasted_iota(jnp.int32, (QP, RP), 1)
    pad = jnp.logical_or(row >= Q, col >= L)
    padded = jnp.where(pad, -1.0, acc)
    out_ref[0] = padded
    rm_ref[0, 0] = jnp.max(padded, axis=1)


def _scores_tc(logits_pad, pos_maps_pad):
    return pl.pallas_call(
        _score_body,
        grid=(B,),
        in_specs=[
            pl.BlockSpec((1, QP, C), lambda b: (b, 0, 0)),
            pl.BlockSpec((1, RP, C), lambda b: (b, 0, 0)),
        ],
        out_specs=[
            pl.BlockSpec((1, QP, RP), lambda b: (b, 0, 0)),
            pl.BlockSpec((1, 1, QP), lambda b: (b, 0, 0)),
        ],
        out_shape=[
            jax.ShapeDtypeStruct((B, QP, RP), jnp.float32),
            jax.ShapeDtypeStruct((B, 1, QP), jnp.float32),
        ],
        compiler_params=pltpu.CompilerParams(
            dimension_semantics=("arbitrary",)),
    )(logits_pad, pos_maps_pad)


def _sc_body(prob_hbm, rowmax_hbm, boxes_hbm, scale_hbm,
             scores_o, labels_o, boxes_o, tk_o,
             prob_v, boxes_v, scale_v, rowmax_v, chunkmax_v,
             score_st, label_st, qidx_st, boxes_st, tk_st, dma_sem):
    b = lax.axis_index("c") * 16 + lax.axis_index("s")
    lanes = lax.iota(jnp.int32, 16)
    lane0 = lanes == 0
    fz = jnp.zeros((16,), jnp.float32)
    iz = jnp.zeros((16,), jnp.int32)

    cps = [pltpu.make_async_copy(prob_hbm.at[b], prob_v, dma_sem),
           pltpu.make_async_copy(rowmax_hbm.at[b], rowmax_v, dma_sem),
           pltpu.make_async_copy(boxes_hbm.at[b], boxes_v, dma_sem),
           pltpu.make_async_copy(scale_hbm.at[b], scale_v, dma_sem)]
    for c in cps:
        c.start()
    for c in cps:
        c.wait()

    sv = scale_v[...]
    _gd = lax.GatherDimensionNumbers(offset_dims=(), collapsed_slice_dims=(0,),
                                     start_index_map=(0,))
    def _vgather(vec, idx):
        return lax.gather(vec, idx[:, None], _gd, (1,),
                          mode=lax.GatherScatterMode.PROMISE_IN_BOUNDS)
    sw = _vgather(sv, iz)
    sh = _vgather(sv, iz + 1)

    @plsc.parallel_loop(0, NCHUNK, unroll=2)
    def init_chunks(j):
        m = rowmax_v[pl.ds(j * 16, 16)]
        plsc.store_scatter(chunkmax_v, [jnp.full((16,), j, jnp.int32)],
                           jnp.full((16,), jnp.max(m)), mask=lane0)
    cpad = lax.iota(jnp.int32, 16) + 48
    plsc.store_scatter(chunkmax_v, [cpad], jnp.full((16,), -3.0),
                       mask=cpad >= NCHUNK)
    qidx_st[pl.ds(288, 16)] = iz

    BIGIV = jnp.full((16,), jnp.int32(1 << 30))

    def extract(i, _):
        cvs = [chunkmax_v[pl.ds(j * 16, 16)] for j in range(4)]
        gmax = jnp.max(jnp.maximum(jnp.maximum(cvs[0], cvs[1]),
                                   jnp.maximum(cvs[2], cvs[3])))
        gmv = jnp.full((16,), gmax)
        jsv = BIGIV
        for j in range(4):
            f = plsc.all_reduce_ffs(cvs[j] == gmv)
            jsv = jnp.minimum(jsv, jnp.where(f >= 16, BIGIV, f + j * 16))
        rv = plsc.load_gather(rowmax_v, [jsv * 16 + lanes])
        qoffv = plsc.all_reduce_ffs(rv == gmv)
        qsv = jsv * 16 + qoffv
        basev = qsv * RP
        lsv = BIGIV
        vs = []
        for cc in range(RP // 16):
            v = plsc.load_gather(prob_v, [basev + cc * 16 + lanes])
            vs.append(v)
            f = plsc.all_reduce_ffs(v == gmv)
            lsv = jnp.minimum(lsv, jnp.where(f >= 16, BIGIV, f + cc * 16))

        bi = jnp.full((16,), i, jnp.int32)
        plsc.store_scatter(score_st, [bi], gmv, mask=lane0)
        plsc.store_scatter(label_st, [bi], lsv, mask=lane0)
        plsc.store_scatter(qidx_st, [bi], qsv, mask=lane0)

        plsc.store_scatter(prob_v, [basev + lsv],
                           jnp.full((16,), -1.0), mask=lane0)
        nm = jnp.full((16,), -1.0)
        for cc in range(RP // 16):
            nm = jnp.maximum(
                nm, jnp.where(cc * 16 + lanes == lsv, jnp.full((16,), -1.0),
                              vs[cc]))
        newrowv = jnp.full((16,), jnp.max(nm))
        plsc.store_scatter(rowmax_v, [qsv], newrowv, mask=lane0)
        newchunk = jnp.max(jnp.where(lanes == qoffv, newrowv, rv))
        plsc.store_scatter(chunkmax_v, [jsv],
                           jnp.full((16,), newchunk), mask=lane0)
        return 0
    lax.fori_loop(0, NSEL, extract, 0)

    @plsc.parallel_loop(0, SP // 16, unroll=2)
    def box_chunk(j):
        k16 = j * 16 + lanes
        valid = k16 < NSEL
        q16 = qidx_st[pl.ds(j * 16, 16)]
        cx = plsc.load_gather(boxes_v, [q16 * 4])
        cy = plsc.load_gather(boxes_v, [q16 * 4 + 1])
        w = plsc.load_gather(boxes_v, [q16 * 4 + 2])
        h = plsc.load_gather(boxes_v, [q16 * 4 + 3])
        plsc.store_scatter(boxes_st, [k16 * 4], (cx - 0.5 * w) * sw,
                           mask=valid)
        plsc.store_scatter(boxes_st, [k16 * 4 + 1], (cy - 0.5 * h) * sh,
                           mask=valid)
        plsc.store_scatter(boxes_st, [k16 * 4 + 2], (cx + 0.5 * w) * sw,
                           mask=valid)
        plsc.store_scatter(boxes_st, [k16 * 4 + 3], (cy + 0.5 * h) * sh,
                           mask=valid)

    @plsc.parallel_loop(0, SP // 16, unroll=2)
    def restore(j):
        k16 = j * 16 + lanes
        valid = k16 < NSEL
        v = score_st[pl.ds(j * 16, 16)]
        q16 = qidx_st[pl.ds(j * 16, 16)]
        l16 = label_st[pl.ds(j * 16, 16)]
        plsc.store_scatter(prob_v, [q16 * RP + l16], v, mask=valid)

    @plsc.parallel_loop(0, NSEL, unroll=2)
    def tkrow(k):
        qv = plsc.load_gather(qidx_st, [jnp.full((16,), k, jnp.int32)])
        basev = qv * RP + lanes
        for cc in range(RP // 16):
            tk_st[pl.ds(k * RP + cc * 16, 16)] = \
                plsc.load_gather(prob_v, [basev + cc * 16])

    cps = [pltpu.make_async_copy(score_st, scores_o.at[b], dma_sem),
           pltpu.make_async_copy(label_st, labels_o.at[b], dma_sem),
           pltpu.make_async_copy(boxes_st, boxes_o.at[b], dma_sem),
           pltpu.make_async_copy(tk_st, tk_o.at[b], dma_sem)]
    for c in cps:
        c.start()
    for c in cps:
        c.wait()


_sc_topk = functools.partial(
    pl.kernel,
    out_type=[
        jax.ShapeDtypeStruct((B, SP), jnp.float32),
        jax.ShapeDtypeStruct((B, SP), jnp.int32),
        jax.ShapeDtypeStruct((B, SP * 4), jnp.float32),
        jax.ShapeDtypeStruct((B, NSEL * RP), jnp.float32),
    ],
    mesh=plsc.VectorSubcoreMesh(core_axis_name="c", subcore_axis_name="s"),
    compiler_params=pltpu.CompilerParams(needs_layout_passes=False),
    scratch_types=[
        pltpu.VMEM((FLAT,), jnp.float32),
        pltpu.VMEM((Q * 4,), jnp.float32),
        pltpu.VMEM((16,), jnp.float32),
        pltpu.VMEM((QP,), jnp.float32),
        pltpu.VMEM((64,), jnp.float32),
        pltpu.VMEM((SP,), jnp.float32),
        pltpu.VMEM((SP,), jnp.int32),
        pltpu.VMEM((SP,), jnp.int32),
        pltpu.VMEM((SP * 4,), jnp.float32),
        pltpu.VMEM((NSEL * RP,), jnp.float32),
        pltpu.SemaphoreType.DMA,
    ],
)(_sc_body)


def kernel(pred_logits, pred_boxes, pos_maps, target_sizes):
    logits_pad = jnp.pad(pred_logits, ((0, 0), (0, QP - Q), (0, 0)))
    pm_pad = jnp.pad(pos_maps, ((0, 0), (0, RP - L), (0, 0)))
    prob, rowmax = _scores_tc(logits_pad, pm_pad)
    prob = prob.reshape(B, FLAT)
    rowmax = rowmax.reshape(B, QP)

    w = target_sizes[:, 1].astype(jnp.float32)
    h = target_sizes[:, 0].astype(jnp.float32)
    scale = jnp.concatenate(
        [w[:, None], h[:, None], jnp.zeros((B, 14), jnp.float32)], axis=1)

    scores_p, labels_p, boxes_p, tk_p = _sc_topk(
        prob, rowmax, pred_boxes.reshape(B, Q * 4), scale)

    scores = scores_p[:, :NSEL]
    labels = labels_p[:, :NSEL]
    boxes = boxes_p[:, :NSEL * 4].reshape(B, NSEL, 4)
    topk_prob = tk_p.reshape(B, NSEL, RP)[:, :, :L]
    return scores, labels, boxes, topk_prob


# speedup vs baseline: 14.5309x; 1.0294x over previous
import functools

import jax
import jax.numpy as jnp
from jax import lax
from jax.experimental import pallas as pl
from jax.experimental.pallas import tpu as pltpu
from jax.experimental.pallas import tpu_sc as plsc

B, Q, C, L = 32, 900, 256, 92
QP = 912
RP = 96
NSEL = 300
SP = 304
NCHUNK = QP // 16
FLAT = QP * RP


def _score_body(logits_ref, pm_ref, out_ref, rm_ref):
    sig = jax.nn.sigmoid(logits_ref[0])
    sig = jnp.concatenate(
        [sig, jnp.zeros((QP - Q, C), jnp.float32)], axis=0)
    pm = pm_ref[0]
    acc = lax.dot_general(sig, pm, (((1,), (1,)), ((), ())),
                          preferred_element_type=jnp.float32)
    row = lax.broadcasted_iota(jnp.int32, (QP, RP), 0)
    col = lax.broadcasted_iota(jnp.int32, (QP, RP), 1)
    pad = jnp.logical_or(row >= Q, col >= L)
    padded = jnp.where(pad, -1.0, acc)
    out_ref[0] = padded
    rm_ref[0, 0] = jnp.max(padded, axis=1)


def _scores_tc(logits_pad, pos_maps_pad):
    return pl.pallas_call(
        _score_body,
        grid=(B,),
        in_specs=[
            pl.BlockSpec((1, Q, C), lambda b: (b, 0, 0)),
            pl.BlockSpec((1, RP, C), lambda b: (b, 0, 0)),
        ],
        out_specs=[
            pl.BlockSpec((1, QP, RP), lambda b: (b, 0, 0)),
            pl.BlockSpec((1, 1, QP), lambda b: (b, 0, 0)),
        ],
        out_shape=[
            jax.ShapeDtypeStruct((B, QP, RP), jnp.float32),
            jax.ShapeDtypeStruct((B, 1, QP), jnp.float32),
        ],
        compiler_params=pltpu.CompilerParams(
            dimension_semantics=("arbitrary",)),
    )(logits_pad, pos_maps_pad)


def _sc_body(prob_hbm, rowmax_hbm, boxes_hbm, scale_hbm,
             scores_o, labels_o, boxes_o, tk_o,
             prob_v, boxes_v, scale_v, rowmax_v, chunkmax_v,
             score_st, label_st, qidx_st, boxes_st, tk_st, dma_sem):
    b = lax.axis_index("c") * 16 + lax.axis_index("s")
    lanes = lax.iota(jnp.int32, 16)
    lane0 = lanes == 0
    fz = jnp.zeros((16,), jnp.float32)
    iz = jnp.zeros((16,), jnp.int32)

    cps = [pltpu.make_async_copy(prob_hbm.at[b], prob_v, dma_sem),
           pltpu.make_async_copy(rowmax_hbm.at[b], rowmax_v, dma_sem),
           pltpu.make_async_copy(boxes_hbm.at[b], boxes_v, dma_sem),
           pltpu.make_async_copy(scale_hbm.at[b], scale_v, dma_sem)]
    for c in cps:
        c.start()
    for c in cps:
        c.wait()

    sv = scale_v[...]
    _gd = lax.GatherDimensionNumbers(offset_dims=(), collapsed_slice_dims=(0,),
                                     start_index_map=(0,))
    def _vgather(vec, idx):
        return lax.gather(vec, idx[:, None], _gd, (1,),
                          mode=lax.GatherScatterMode.PROMISE_IN_BOUNDS)
    sw = _vgather(sv, iz)
    sh = _vgather(sv, iz + 1)

    @plsc.parallel_loop(0, NCHUNK, unroll=2)
    def init_chunks(j):
        m = rowmax_v[pl.ds(j * 16, 16)]
        plsc.store_scatter(chunkmax_v, [jnp.full((16,), j, jnp.int32)],
                           jnp.full((16,), jnp.max(m)), mask=lane0)
    cpad = lax.iota(jnp.int32, 16) + 48
    plsc.store_scatter(chunkmax_v, [cpad], jnp.full((16,), -3.0),
                       mask=cpad >= NCHUNK)
    qidx_st[pl.ds(288, 16)] = iz

    BIGIV = jnp.full((16,), jnp.int32(1 << 30))

    def extract(i, cvs):
        gmax = jnp.max(jnp.maximum(jnp.maximum(cvs[0], cvs[1]),
                                   jnp.maximum(cvs[2], cvs[3])))
        gmv = jnp.full((16,), gmax)
        jsv = BIGIV
        for j in range(4):
            f = plsc.all_reduce_ffs(cvs[j] == gmv)
            jsv = jnp.minimum(jsv, jnp.where(f >= 16, BIGIV, f + j * 16))
        rv = plsc.load_gather(rowmax_v, [jsv * 16 + lanes])
        qoffv = plsc.all_reduce_ffs(rv == gmv)
        qsv = jsv * 16 + qoffv
        basev = qsv * RP
        lsv = BIGIV
        vs = []
        for cc in range(RP // 16):
            v = plsc.load_gather(prob_v, [basev + cc * 16 + lanes])
            vs.append(v)
            f = plsc.all_reduce_ffs(v == gmv)
            lsv = jnp.minimum(lsv, jnp.where(f >= 16, BIGIV, f + cc * 16))

        bi = jnp.full((16,), i, jnp.int32)
        plsc.store_scatter(score_st, [bi], gmv, mask=lane0)
        plsc.store_scatter(label_st, [bi], lsv, mask=lane0)
        plsc.store_scatter(qidx_st, [bi], qsv, mask=lane0)

        plsc.store_scatter(prob_v, [basev + lsv],
                           jnp.full((16,), -1.0), mask=lane0)
        nm = jnp.full((16,), -1.0)
        for cc in range(RP // 16):
            nm = jnp.maximum(
                nm, jnp.where(cc * 16 + lanes == lsv, jnp.full((16,), -1.0),
                              vs[cc]))
        newrowv = jnp.full((16,), jnp.max(nm))
        plsc.store_scatter(rowmax_v, [qsv], newrowv, mask=lane0)
        newchunkv = jnp.full((16,), jnp.max(jnp.where(lanes == qoffv,
                                                      newrowv, rv)))
        jv, lv = jsv >> 4, jsv & 15
        return tuple(
            jnp.where(jnp.logical_and(jv == j, lanes == lv), newchunkv, c)
            for j, c in enumerate(cvs))
    lax.fori_loop(0, NSEL, extract,
                  tuple(chunkmax_v[pl.ds(j * 16, 16)] for j in range(4)))

    @plsc.parallel_loop(0, SP // 16, unroll=2)
    def box_chunk(j):
        k16 = j * 16 + lanes
        valid = k16 < NSEL
        q16 = qidx_st[pl.ds(j * 16, 16)]
        cx = plsc.load_gather(boxes_v, [q16 * 4])
        cy = plsc.load_gather(boxes_v, [q16 * 4 + 1])
        w = plsc.load_gather(boxes_v, [q16 * 4 + 2])
        h = plsc.load_gather(boxes_v, [q16 * 4 + 3])
        plsc.store_scatter(boxes_st, [k16 * 4], (cx - 0.5 * w) * sw,
                           mask=valid)
        plsc.store_scatter(boxes_st, [k16 * 4 + 1], (cy - 0.5 * h) * sh,
                           mask=valid)
        plsc.store_scatter(boxes_st, [k16 * 4 + 2], (cx + 0.5 * w) * sw,
                           mask=valid)
        plsc.store_scatter(boxes_st, [k16 * 4 + 3], (cy + 0.5 * h) * sh,
                           mask=valid)

    @plsc.parallel_loop(0, SP // 16, unroll=2)
    def restore(j):
        k16 = j * 16 + lanes
        valid = k16 < NSEL
        v = score_st[pl.ds(j * 16, 16)]
        q16 = qidx_st[pl.ds(j * 16, 16)]
        l16 = label_st[pl.ds(j * 16, 16)]
        plsc.store_scatter(prob_v, [q16 * RP + l16], v, mask=valid)

    @plsc.parallel_loop(0, NSEL, unroll=2)
    def tkrow(k):
        qv = plsc.load_gather(qidx_st, [jnp.full((16,), k, jnp.int32)])
        basev = qv * RP + lanes
        for cc in range(RP // 16):
            tk_st[pl.ds(k * RP + cc * 16, 16)] = \
                plsc.load_gather(prob_v, [basev + cc * 16])

    cps = [pltpu.make_async_copy(score_st, scores_o.at[b], dma_sem),
           pltpu.make_async_copy(label_st, labels_o.at[b], dma_sem),
           pltpu.make_async_copy(boxes_st, boxes_o.at[b], dma_sem),
           pltpu.make_async_copy(tk_st, tk_o.at[b], dma_sem)]
    for c in cps:
        c.start()
    for c in cps:
        c.wait()


_sc_topk = functools.partial(
    pl.kernel,
    out_type=[
        jax.ShapeDtypeStruct((B, SP), jnp.float32),
        jax.ShapeDtypeStruct((B, SP), jnp.int32),
        jax.ShapeDtypeStruct((B, SP * 4), jnp.float32),
        jax.ShapeDtypeStruct((B, NSEL * RP), jnp.float32),
    ],
    mesh=plsc.VectorSubcoreMesh(core_axis_name="c", subcore_axis_name="s"),
    compiler_params=pltpu.CompilerParams(needs_layout_passes=False),
    scratch_types=[
        pltpu.VMEM((FLAT,), jnp.float32),
        pltpu.VMEM((Q * 4,), jnp.float32),
        pltpu.VMEM((16,), jnp.float32),
        pltpu.VMEM((QP,), jnp.float32),
        pltpu.VMEM((64,), jnp.float32),
        pltpu.VMEM((SP,), jnp.float32),
        pltpu.VMEM((SP,), jnp.int32),
        pltpu.VMEM((SP,), jnp.int32),
        pltpu.VMEM((SP * 4,), jnp.float32),
        pltpu.VMEM((NSEL * RP,), jnp.float32),
        pltpu.SemaphoreType.DMA,
    ],
)(_sc_body)


def kernel(pred_logits, pred_boxes, pos_maps, target_sizes):
    pm_pad = jnp.pad(pos_maps, ((0, 0), (0, RP - L), (0, 0)))
    prob, rowmax = _scores_tc(pred_logits, pm_pad)
    prob = prob.reshape(B, FLAT)
    rowmax = rowmax.reshape(B, QP)

    w = target_sizes[:, 1].astype(jnp.float32)
    h = target_sizes[:, 0].astype(jnp.float32)
    scale = jnp.concatenate(
        [w[:, None], h[:, None], jnp.zeros((B, 14), jnp.float32)], axis=1)

    scores_p, labels_p, boxes_p, tk_p = _sc_topk(
        prob, rowmax, pred_boxes.reshape(B, Q * 4), scale)

    scores = scores_p[:, :NSEL]
    labels = labels_p[:, :NSEL]
    boxes = boxes_p[:, :NSEL * 4].reshape(B, NSEL, 4)
    topk_prob = tk_p.reshape(B, NSEL, RP)[:, :, :L]
    return scores, labels, boxes, topk_prob
